# baseline (speedup 1.0000x reference)
import functools

import jax
import jax.numpy as jnp
from jax.experimental import pallas as pl
from jax.experimental.pallas import tpu as pltpu

_NA = 3
_NC = 3
_G = 152
_SXY = 1.05
_OFF = 0.5 * (_SXY - 1.0)
_AW = (1.08, 3.42, 6.63)
_AH = (1.19, 4.41, 11.38)
_R = 8


def _decode_body(stride_ref, x_ref, o_ref):
    p = x_ref[0]
    stride = stride_ref[0]
    i = pl.program_id(0)
    j = pl.program_id(1)
    a = jax.lax.rem(i, _NA)

    aw = jnp.where(a == 0, _AW[0], jnp.where(a == 1, _AW[1], _AW[2]))
    ah = jnp.where(a == 0, _AH[0], jnp.where(a == 1, _AH[1], _AH[2]))

    gx = jax.lax.broadcasted_iota(jnp.int32, (_R, _G), 1).astype(jnp.float32)
    gy = (j * _R).astype(jnp.float32) + jax.lax.broadcasted_iota(
        jnp.int32, (_R, _G), 0).astype(jnp.float32)

    o0 = (jax.nn.sigmoid(p[0]) * _SXY - _OFF + gx) * stride
    o1 = (jax.nn.sigmoid(p[1]) * _SXY - _OFF + gy) * stride
    o2 = jnp.exp(p[2]) * aw
    o3 = jnp.exp(p[3]) * ah
    o4 = jax.nn.sigmoid(p[4])
    o5 = jax.nn.sigmoid(p[5])
    d0, d1, d2, d3 = p[6], p[7], p[8], p[9]
    idx = jnp.where(d1 > d0, 1.0, 0.0)
    best = jnp.maximum(d0, d1)
    idx = jnp.where(d2 > best, 2.0, idx)
    best = jnp.maximum(best, d2)
    o6 = jnp.where(d3 > best, 3.0, idx)
    o7 = jax.nn.sigmoid(p[10])
    o8 = jax.nn.sigmoid(p[11])
    o9 = jax.nn.sigmoid(p[12])
    o10 = jax.nn.sigmoid(p[13])

    vals = jnp.stack([o0, o1, o2, o3, o4, o5, o6, o7, o8, o9, o10], axis=-1)
    o_ref[0] = vals


@functools.partial(jax.jit, static_argnums=())
def kernel(x, img_size):
    n = x.shape[0]
    xr = x.reshape(n * _NA, _NC + 11, _G, _G)
    stride = (jnp.float32(img_size) / _G).reshape(1)

    out = pl.pallas_call(
        _decode_body,
        grid=(n * _NA, _G // _R),
        in_specs=[
            pl.BlockSpec(memory_space=pltpu.SMEM),
            pl.BlockSpec((1, _NC + 11, _R, _G), lambda i, j: (i, 0, j, 0)),
        ],
        out_specs=pl.BlockSpec((1, _R, _G, 11), lambda i, j: (i, j, 0, 0)),
        out_shape=jax.ShapeDtypeStruct((n * _NA, _G, _G, 11), jnp.float32),
        compiler_params=pltpu.CompilerParams(
            dimension_semantics=("arbitrary", "arbitrary"),
        ),
    )(stride, xr)
    return out.reshape(n, _NA * _G * _G, 11)


# speedup vs baseline: 4.1705x; 4.1705x over previous
import functools

import jax
import jax.numpy as jnp
from jax.experimental import pallas as pl
from jax.experimental.pallas import tpu as pltpu

_NA = 3
_NC = 3
_G = 152
_GG = _G * _G
_SXY = 1.05
_OFF = 0.5 * (_SXY - 1.0)
_AW = (1.08, 3.42, 6.63)
_AH = (1.19, 4.41, 11.38)
_B = _GG


def _decode_body(stride_ref, x_ref, o_ref):
    p = x_ref[0]
    stride = stride_ref[0]
    i = pl.program_id(0)
    a = jax.lax.rem(i, _NA)

    aw = jnp.where(a == 0, _AW[0], jnp.where(a == 1, _AW[1], _AW[2]))
    ah = jnp.where(a == 0, _AH[0], jnp.where(a == 1, _AH[1], _AH[2]))

    lane = jax.lax.broadcasted_iota(jnp.int32, (1, _B), 1)
    q = jnp.floor(lane.astype(jnp.float32) * (1.0 / _G)).astype(jnp.int32)
    r = lane - q * _G
    q = q + (r >= _G).astype(jnp.int32) - (r < 0).astype(jnp.int32)
    r = r - _G * (r >= _G).astype(jnp.int32) + _G * (r < 0).astype(jnp.int32)
    gx = r.astype(jnp.float32)
    gy = q.astype(jnp.float32)

    row = jax.lax.broadcasted_iota(jnp.int32, (14, _B), 0)
    sig = jax.nn.sigmoid(p)
    ex = jnp.exp(p)
    sig_s = pltpu.roll(sig, shift=11, axis=0)

    d6, d7, d8, d9 = p[6], p[7], p[8], p[9]
    idx = jnp.where(d7 > d6, 1.0, 0.0)
    best = jnp.maximum(d6, d7)
    idx = jnp.where(d8 > best, 2.0, idx)
    best = jnp.maximum(best, d8)
    dirv = jnp.where(d9 > best, 3.0, idx)

    g = jnp.where(row == 0, gx, gy)
    anch = jnp.where(row == 2, aw, ah)
    val = jnp.where(row < 2, (sig * _SXY - _OFF + g) * stride,
          jnp.where(row < 4, ex * anch,
          jnp.where(row == 6, dirv[None, :],
          jnp.where(row < 6, sig, sig_s))))

    t = val.T
    o_ref[0] = t[:, :11]


@functools.partial(jax.jit, static_argnums=())
def kernel(x, img_size):
    n = x.shape[0]
    xr = x.reshape(n * _NA, _NC + 11, _GG)
    stride = (jnp.float32(img_size) / _G).reshape(1)

    out = pl.pallas_call(
        _decode_body,
        grid=(n * _NA,),
        in_specs=[
            pl.BlockSpec(memory_space=pltpu.SMEM),
            pl.BlockSpec((1, _NC + 11, _B), lambda i: (i, 0, 0)),
        ],
        out_specs=pl.BlockSpec((1, _B, 11), lambda i: (i, 0, 0)),
        out_shape=jax.ShapeDtypeStruct((n * _NA, _GG, 11), jnp.float32),
        compiler_params=pltpu.CompilerParams(
            dimension_semantics=("arbitrary",),
        ),
    )(stride, xr)
    return out.reshape(n, _NA * _GG, 11)
